# baseline (speedup 1.0000x reference)
import functools

import jax
import jax.numpy as jnp
from jax import lax
from jax.experimental import pallas as pl
from jax.experimental.pallas import tpu as pltpu
from jax.experimental.pallas import tpu_sc as plsc

BATCH = 16384
ENT_D = 128
REL_D = 64
NUM_BUCKETS = 52
SECONDS_PER_WEEK = 7 * 86400

NUM_CORES = 2
NUM_SUBCORES = 16
NUM_WORKERS = NUM_CORES * NUM_SUBCORES
ROWS_PER_WORKER = BATCH // NUM_WORKERS
CHUNK = 128
NUM_CHUNKS = ROWS_PER_WORKER // CHUNK
LANES = 16
GROUPS = CHUNK // LANES
NVEC = ENT_D // LANES
INV_WEEK = 1.0 / SECONDS_PER_WEEK


def _rel_cos_sin(relation_table):
    def body(r_ref, o_ref):
        r = r_ref[...]
        o_ref[:, :REL_D] = jnp.cos(r)
        o_ref[:, REL_D:] = jnp.sin(r)

    return pl.pallas_call(
        body,
        out_shape=jax.ShapeDtypeStruct((relation_table.shape[0], ENT_D),
                                       jnp.float32),
    )(relation_table)


def _rsqrt(m):
    i = lax.bitcast_convert_type(m, jnp.int32)
    i = jnp.int32(0x5F3759DF) - lax.shift_right_arithmetic(i, 1)
    y = lax.bitcast_convert_type(i, jnp.float32)
    hm = m * jnp.float32(0.5)
    for _ in range(2):
        y = y * (jnp.float32(1.5) - hm * y * y)
    return y


def _sc_score(entity_table, rel_cs_table, time_normals,
              head_idx, tail_idx, relation_idx, timestamps):
    mesh = plsc.VectorSubcoreMesh(core_axis_name="c", subcore_axis_name="s")

    @functools.partial(
        pl.kernel,
        mesh=mesh,
        out_type=jax.ShapeDtypeStruct((BATCH,), jnp.float32),
        compiler_params=pltpu.CompilerParams(needs_layout_passes=False),
        scratch_types=[
            pltpu.VMEM((CHUNK,), jnp.int32),
            pltpu.VMEM((CHUNK,), jnp.int32),
            pltpu.VMEM((CHUNK,), jnp.int32),
            pltpu.VMEM((CHUNK,), jnp.int32),
            pltpu.VMEM((CHUNK,), jnp.int32),
            pltpu.VMEM((CHUNK, ENT_D), jnp.float32),
            pltpu.VMEM((CHUNK, ENT_D), jnp.float32),
            pltpu.VMEM((CHUNK, ENT_D), jnp.float32),
            pltpu.VMEM((CHUNK, ENT_D), jnp.float32),
            pltpu.VMEM((ROWS_PER_WORKER,), jnp.float32),
            pltpu.SemaphoreType.DMA,
        ],
    )
    def score_kernel(ent_hbm, rel_hbm, tn_hbm, hidx_hbm, tidx_hbm, ridx_hbm,
                     ts_hbm, out_hbm,
                     hidx_v, tidx_v, ridx_v, ts_v, bidx_v, h_v, t_v, r_v,
                     n_v, out_v, sem):
        wid = lax.axis_index("s") * NUM_CORES + lax.axis_index("c")
        lane_iota = lax.iota(jnp.int32, LANES)

        def chunk_body(g, _):
            base = wid * ROWS_PER_WORKER + g * CHUNK
            pltpu.sync_copy(hidx_hbm.at[pl.ds(base, CHUNK)], hidx_v)
            pltpu.sync_copy(tidx_hbm.at[pl.ds(base, CHUNK)], tidx_v)
            pltpu.sync_copy(ridx_hbm.at[pl.ds(base, CHUNK)], ridx_v)
            pltpu.sync_copy(ts_hbm.at[pl.ds(base, CHUNK)], ts_v)
            ch = pltpu.async_copy(ent_hbm.at[hidx_v], h_v, sem)
            ct = pltpu.async_copy(ent_hbm.at[tidx_v], t_v, sem)
            cr = pltpu.async_copy(rel_hbm.at[ridx_v], r_v, sem)
            for jj in range(GROUPS):
                ts16 = ts_v[pl.ds(jj * LANES, LANES)]
                q = (ts16.astype(jnp.float32)
                     * jnp.float32(INV_WEEK)).astype(jnp.int32)
                rem = ts16 - q * SECONDS_PER_WEEK
                q = q + jnp.where(rem < 0, -1, 0)
                q = q + jnp.where(rem >= SECONDS_PER_WEEK, 1, 0)
                bidx_v[pl.ds(jj * LANES, LANES)] = jnp.minimum(
                    q, NUM_BUCKETS - 1)
            cn = pltpu.async_copy(tn_hbm.at[bidx_v], n_v, sem)
            ch.wait()
            ct.wait()
            cr.wait()
            cn.wait()

            def group_body(j, _):
                acc_out = jnp.zeros((LANES,), jnp.float32)
                for e in range(LANES):
                    i = j * LANES + e
                    dph = jnp.float32(0.0)
                    dpt = jnp.float32(0.0)
                    for k in range(NVEC):
                        sl = pl.ds(k * LANES, LANES)
                        n = n_v[i, sl]
                        dph = dph + jnp.sum(h_v[i, sl] * n)
                        dpt = dpt + jnp.sum(t_v[i, sl] * n)
                    acc = jnp.zeros((LANES,), jnp.float32)
                    for k in range(NVEC // 2):
                        slr = pl.ds(k * LANES, LANES)
                        sli = pl.ds(REL_D + k * LANES, LANES)
                        nre = n_v[i, slr]
                        nim = n_v[i, sli]
                        hpre = h_v[i, slr] - dph * nre
                        hpim = h_v[i, sli] - dph * nim
                        tpre = t_v[i, slr] - dpt * nre
                        tpim = t_v[i, sli] - dpt * nim
                        c = r_v[i, slr]
                        s = r_v[i, sli]
                        re_s = hpre * c - hpim * s - tpre
                        im_s = hpre * s + hpim * c - tpim
                        m = re_s * re_s + im_s * im_s + jnp.float32(1e-12)
                        acc = acc + m * _rsqrt(m)
                    total = jnp.sum(acc)
                    acc_out = jnp.where(lane_iota == e, -total, acc_out)
                out_v[pl.ds(g * CHUNK + j * LANES, LANES)] = acc_out
                return 0

            lax.fori_loop(0, GROUPS, group_body, 0, unroll=False)
            return 0

        lax.fori_loop(0, NUM_CHUNKS, chunk_body, 0, unroll=False)
        pltpu.sync_copy(out_v, out_hbm.at[pl.ds(wid * ROWS_PER_WORKER,
                                                ROWS_PER_WORKER)])

    return score_kernel(entity_table, rel_cs_table, time_normals,
                        head_idx, tail_idx, relation_idx, timestamps)


def kernel(head_idx, relation_idx, tail_idx, timestamps,
           entity_table, relation_table, time_normals):
    rel_cs = _rel_cos_sin(relation_table)
    return _sc_score(entity_table, rel_cs, time_normals,
                     head_idx.astype(jnp.int32), tail_idx.astype(jnp.int32),
                     relation_idx.astype(jnp.int32),
                     timestamps.astype(jnp.int32))


# speedup vs baseline: 1.6406x; 1.6406x over previous
import functools

import jax
import jax.numpy as jnp
from jax import lax
from jax.experimental import pallas as pl
from jax.experimental.pallas import tpu as pltpu
from jax.experimental.pallas import tpu_sc as plsc

BATCH = 16384
ENT_D = 128
REL_D = 64
NUM_BUCKETS = 52
SECONDS_PER_WEEK = 7 * 86400

NUM_CORES = 2
NUM_SUBCORES = 16
NUM_WORKERS = NUM_CORES * NUM_SUBCORES
ROWS_PER_WORKER = BATCH // NUM_WORKERS
CHUNK = 128
NUM_CHUNKS = ROWS_PER_WORKER // CHUNK
LANES = 16
GROUPS = CHUNK // LANES
NVEC = ENT_D // LANES
INV_WEEK = 1.0 / SECONDS_PER_WEEK


def _rel_cos_sin(relation_table):
    def body(r_ref, o_ref):
        r = r_ref[...]
        o_ref[:, :REL_D] = jnp.cos(r)
        o_ref[:, REL_D:] = jnp.sin(r)

    return pl.pallas_call(
        body,
        out_shape=jax.ShapeDtypeStruct((relation_table.shape[0], ENT_D),
                                       jnp.float32),
    )(relation_table)


def _rsqrt(m):
    i = lax.bitcast_convert_type(m, jnp.int32)
    i = jnp.int32(0x5F3759DF) - lax.shift_right_arithmetic(i, 1)
    y = lax.bitcast_convert_type(i, jnp.float32)
    hm = m * jnp.float32(0.5)
    for _ in range(2):
        y = y * (jnp.float32(1.5) - hm * y * y)
    return y


def _sc_score(entity_table, rel_cs_table, time_normals,
              head_idx, tail_idx, relation_idx, timestamps):
    mesh = plsc.VectorSubcoreMesh(core_axis_name="c", subcore_axis_name="s")

    @functools.partial(
        pl.kernel,
        mesh=mesh,
        out_type=jax.ShapeDtypeStruct((BATCH,), jnp.float32),
        compiler_params=pltpu.CompilerParams(needs_layout_passes=False),
        scratch_types=[
            pltpu.VMEM((CHUNK,), jnp.int32),
            pltpu.VMEM((CHUNK,), jnp.int32),
            pltpu.VMEM((CHUNK,), jnp.int32),
            pltpu.VMEM((CHUNK,), jnp.int32),
            pltpu.VMEM((CHUNK,), jnp.int32),
            pltpu.VMEM((CHUNK, ENT_D), jnp.float32),
            pltpu.VMEM((CHUNK, ENT_D), jnp.float32),
            pltpu.VMEM((CHUNK, ENT_D), jnp.float32),
            pltpu.VMEM((CHUNK, ENT_D), jnp.float32),
            pltpu.VMEM((ROWS_PER_WORKER,), jnp.float32),
            pltpu.SemaphoreType.DMA,
        ],
    )
    def score_kernel(ent_hbm, rel_hbm, tn_hbm, hidx_hbm, tidx_hbm, ridx_hbm,
                     ts_hbm, out_hbm,
                     hidx_v, tidx_v, ridx_v, ts_v, bidx_v, h_v, t_v, r_v,
                     n_v, out_v, sem):
        wid = lax.axis_index("s") * NUM_CORES + lax.axis_index("c")
        lane_iota = lax.iota(jnp.int32, LANES)

        def chunk_body(g, _):
            base = wid * ROWS_PER_WORKER + g * CHUNK
            pltpu.sync_copy(hidx_hbm.at[pl.ds(base, CHUNK)], hidx_v)
            pltpu.sync_copy(tidx_hbm.at[pl.ds(base, CHUNK)], tidx_v)
            pltpu.sync_copy(ridx_hbm.at[pl.ds(base, CHUNK)], ridx_v)
            pltpu.sync_copy(ts_hbm.at[pl.ds(base, CHUNK)], ts_v)
            ch = pltpu.async_copy(ent_hbm.at[hidx_v], h_v, sem)
            ct = pltpu.async_copy(ent_hbm.at[tidx_v], t_v, sem)
            cr = pltpu.async_copy(rel_hbm.at[ridx_v], r_v, sem)
            for jj in range(GROUPS):
                ts16 = ts_v[pl.ds(jj * LANES, LANES)]
                q = (ts16.astype(jnp.float32)
                     * jnp.float32(INV_WEEK)).astype(jnp.int32)
                rem = ts16 - q * SECONDS_PER_WEEK
                q = q + jnp.where(rem < 0, -1, 0)
                q = q + jnp.where(rem >= SECONDS_PER_WEEK, 1, 0)
                bidx_v[pl.ds(jj * LANES, LANES)] = jnp.minimum(
                    q, NUM_BUCKETS - 1)
            cn = pltpu.async_copy(tn_hbm.at[bidx_v], n_v, sem)
            ch.wait()
            ct.wait()
            cr.wait()
            cn.wait()

            def group_body(j, _):
                acc_out = jnp.zeros((LANES,), jnp.float32)
                for e in range(LANES):
                    i = j * LANES + e
                    dph_vec = jnp.zeros((LANES,), jnp.float32)
                    dpt_vec = jnp.zeros((LANES,), jnp.float32)
                    for k in range(NVEC):
                        sl = pl.ds(k * LANES, LANES)
                        n = n_v[i, sl]
                        dph_vec = dph_vec + h_v[i, sl] * n
                        dpt_vec = dpt_vec + t_v[i, sl] * n
                    dph = jnp.sum(dph_vec)
                    dpt = jnp.sum(dpt_vec)
                    acc = jnp.zeros((LANES,), jnp.float32)
                    for k in range(NVEC // 2):
                        slr = pl.ds(k * LANES, LANES)
                        sli = pl.ds(REL_D + k * LANES, LANES)
                        nre = n_v[i, slr]
                        nim = n_v[i, sli]
                        hpre = h_v[i, slr] - dph * nre
                        hpim = h_v[i, sli] - dph * nim
                        tpre = t_v[i, slr] - dpt * nre
                        tpim = t_v[i, sli] - dpt * nim
                        c = r_v[i, slr]
                        s = r_v[i, sli]
                        re_s = hpre * c - hpim * s - tpre
                        im_s = hpre * s + hpim * c - tpim
                        m = re_s * re_s + im_s * im_s + jnp.float32(1e-12)
                        acc = acc + m * _rsqrt(m)
                    total = jnp.sum(acc)
                    acc_out = jnp.where(lane_iota == e, -total, acc_out)
                out_v[pl.ds(g * CHUNK + j * LANES, LANES)] = acc_out
                return 0

            lax.fori_loop(0, GROUPS, group_body, 0, unroll=False)
            return 0

        lax.fori_loop(0, NUM_CHUNKS, chunk_body, 0, unroll=False)
        pltpu.sync_copy(out_v, out_hbm.at[pl.ds(wid * ROWS_PER_WORKER,
                                                ROWS_PER_WORKER)])

    return score_kernel(entity_table, rel_cs_table, time_normals,
                        head_idx, tail_idx, relation_idx, timestamps)


def kernel(head_idx, relation_idx, tail_idx, timestamps,
           entity_table, relation_table, time_normals):
    rel_cs = _rel_cos_sin(relation_table)
    return _sc_score(entity_table, rel_cs, time_normals,
                     head_idx.astype(jnp.int32), tail_idx.astype(jnp.int32),
                     relation_idx.astype(jnp.int32),
                     timestamps.astype(jnp.int32))


# speedup vs baseline: 1.8590x; 1.1331x over previous
import functools

import jax
import jax.numpy as jnp
from jax import lax
from jax.experimental import pallas as pl
from jax.experimental.pallas import tpu as pltpu
from jax.experimental.pallas import tpu_sc as plsc

BATCH = 16384
ENT_D = 128
REL_D = 64
NUM_BUCKETS = 52
SECONDS_PER_WEEK = 7 * 86400

NUM_CORES = 2
NUM_SUBCORES = 16
NUM_WORKERS = NUM_CORES * NUM_SUBCORES
ROWS_PER_WORKER = BATCH // NUM_WORKERS
CHUNK = 64
NUM_CHUNKS = ROWS_PER_WORKER // CHUNK
NUM_PAIRS = NUM_CHUNKS // 2
LANES = 16
GROUPS = CHUNK // LANES
NVEC = ENT_D // LANES
INV_WEEK = 1.0 / SECONDS_PER_WEEK


def _rel_cos_sin(relation_table):
    def body(r_ref, o_ref):
        r = r_ref[...]
        o_ref[:, :REL_D] = jnp.cos(r)
        o_ref[:, REL_D:] = jnp.sin(r)

    return pl.pallas_call(
        body,
        out_shape=jax.ShapeDtypeStruct((relation_table.shape[0], ENT_D),
                                       jnp.float32),
    )(relation_table)


def _rsqrt(m):
    i = lax.bitcast_convert_type(m, jnp.int32)
    i = jnp.int32(0x5F3759DF) - lax.shift_right_arithmetic(i, 1)
    y = lax.bitcast_convert_type(i, jnp.float32)
    y = y * (jnp.float32(1.5) - (m * jnp.float32(0.5)) * y * y)
    return y


def _sc_score(entity_table, rel_cs_table, time_normals,
              head_idx, tail_idx, relation_idx, timestamps):
    mesh = plsc.VectorSubcoreMesh(core_axis_name="c", subcore_axis_name="s")

    buf = lambda: [
        pltpu.VMEM((CHUNK,), jnp.int32),
        pltpu.VMEM((CHUNK,), jnp.int32),
        pltpu.VMEM((CHUNK,), jnp.int32),
        pltpu.VMEM((CHUNK,), jnp.int32),
        pltpu.VMEM((CHUNK,), jnp.int32),
        pltpu.VMEM((CHUNK, ENT_D), jnp.float32),
        pltpu.VMEM((CHUNK, ENT_D), jnp.float32),
        pltpu.VMEM((CHUNK, ENT_D), jnp.float32),
        pltpu.VMEM((CHUNK, ENT_D), jnp.float32),
        pltpu.SemaphoreType.DMA,
    ]

    @functools.partial(
        pl.kernel,
        mesh=mesh,
        out_type=jax.ShapeDtypeStruct((BATCH,), jnp.float32),
        compiler_params=pltpu.CompilerParams(needs_layout_passes=False),
        scratch_types=buf() + buf() + [
            pltpu.VMEM((ROWS_PER_WORKER,), jnp.float32),
        ],
    )
    def score_kernel(ent_hbm, rel_hbm, tn_hbm, hidx_hbm, tidx_hbm, ridx_hbm,
                     ts_hbm, out_hbm, *bufs):
        (ha, ta, ra, tsa, ba, h_a, t_a, r_a, n_a, sema,
         hb, tb, rb, tsb, bb, h_b, t_b, r_b, n_b, semb, out_v) = bufs
        buf_a = (ha, ta, ra, tsa, ba, h_a, t_a, r_a, n_a, sema)
        buf_b = (hb, tb, rb, tsb, bb, h_b, t_b, r_b, n_b, semb)
        wid = lax.axis_index("s") * NUM_CORES + lax.axis_index("c")
        lane_iota = lax.iota(jnp.int32, LANES)

        def fire(cidx, b):
            hidx_v, tidx_v, ridx_v, ts_v, bidx_v, h_v, t_v, r_v, n_v, sem = b
            base = wid * ROWS_PER_WORKER + cidx * CHUNK
            pltpu.sync_copy(hidx_hbm.at[pl.ds(base, CHUNK)], hidx_v)
            pltpu.sync_copy(tidx_hbm.at[pl.ds(base, CHUNK)], tidx_v)
            pltpu.sync_copy(ridx_hbm.at[pl.ds(base, CHUNK)], ridx_v)
            pltpu.sync_copy(ts_hbm.at[pl.ds(base, CHUNK)], ts_v)
            for jj in range(GROUPS):
                ts16 = ts_v[pl.ds(jj * LANES, LANES)]
                q = (ts16.astype(jnp.float32)
                     * jnp.float32(INV_WEEK)).astype(jnp.int32)
                rem = ts16 - q * SECONDS_PER_WEEK
                q = q + jnp.where(rem < 0, -1, 0)
                q = q + jnp.where(rem >= SECONDS_PER_WEEK, 1, 0)
                bidx_v[pl.ds(jj * LANES, LANES)] = jnp.minimum(
                    q, NUM_BUCKETS - 1)
            pltpu.async_copy(ent_hbm.at[hidx_v], h_v, sem)
            pltpu.async_copy(ent_hbm.at[tidx_v], t_v, sem)
            pltpu.async_copy(rel_hbm.at[ridx_v], r_v, sem)
            pltpu.async_copy(tn_hbm.at[bidx_v], n_v, sem)

        def drain(b):
            hidx_v, tidx_v, ridx_v, ts_v, bidx_v, h_v, t_v, r_v, n_v, sem = b
            pltpu.make_async_copy(ent_hbm.at[hidx_v], h_v, sem).wait()
            pltpu.make_async_copy(ent_hbm.at[tidx_v], t_v, sem).wait()
            pltpu.make_async_copy(rel_hbm.at[ridx_v], r_v, sem).wait()
            pltpu.make_async_copy(tn_hbm.at[bidx_v], n_v, sem).wait()

        def compute(cidx, b):
            _, _, _, _, _, h_v, t_v, r_v, n_v, _ = b

            def group_body(j, _):
                acc_out = jnp.zeros((LANES,), jnp.float32)
                for e in range(LANES):
                    i = j * LANES + e
                    dph_vec = jnp.zeros((LANES,), jnp.float32)
                    dpt_vec = jnp.zeros((LANES,), jnp.float32)
                    for k in range(NVEC):
                        sl = pl.ds(k * LANES, LANES)
                        n = n_v[i, sl]
                        dph_vec = dph_vec + h_v[i, sl] * n
                        dpt_vec = dpt_vec + t_v[i, sl] * n
                    dph = jnp.sum(dph_vec)
                    dpt = jnp.sum(dpt_vec)
                    acc = jnp.zeros((LANES,), jnp.float32)
                    for k in range(NVEC // 2):
                        slr = pl.ds(k * LANES, LANES)
                        sli = pl.ds(REL_D + k * LANES, LANES)
                        nre = n_v[i, slr]
                        nim = n_v[i, sli]
                        hpre = h_v[i, slr] - dph * nre
                        hpim = h_v[i, sli] - dph * nim
                        tpre = t_v[i, slr] - dpt * nre
                        tpim = t_v[i, sli] - dpt * nim
                        c = r_v[i, slr]
                        s = r_v[i, sli]
                        re_s = hpre * c - hpim * s - tpre
                        im_s = hpre * s + hpim * c - tpim
                        m = re_s * re_s + im_s * im_s + jnp.float32(1e-12)
                        acc = acc + m * _rsqrt(m)
                    total = jnp.sum(acc)
                    acc_out = jnp.where(lane_iota == e, -total, acc_out)
                out_v[pl.ds(cidx * CHUNK + j * LANES, LANES)] = acc_out
                return 0

            lax.fori_loop(0, GROUPS, group_body, 0, unroll=False)

        fire(0, buf_a)

        def pair_body(p, _):
            fire(2 * p + 1, buf_b)
            drain(buf_a)
            compute(2 * p, buf_a)

            @pl.when(p < NUM_PAIRS - 1)
            def _():
                fire(2 * p + 2, buf_a)

            drain(buf_b)
            compute(2 * p + 1, buf_b)
            return 0

        lax.fori_loop(0, NUM_PAIRS, pair_body, 0, unroll=False)
        pltpu.sync_copy(out_v, out_hbm.at[pl.ds(wid * ROWS_PER_WORKER,
                                                ROWS_PER_WORKER)])

    return score_kernel(entity_table, rel_cs_table, time_normals,
                        head_idx, tail_idx, relation_idx, timestamps)


def kernel(head_idx, relation_idx, tail_idx, timestamps,
           entity_table, relation_table, time_normals):
    rel_cs = _rel_cos_sin(relation_table)
    return _sc_score(entity_table, rel_cs, time_normals,
                     head_idx.astype(jnp.int32), tail_idx.astype(jnp.int32),
                     relation_idx.astype(jnp.int32),
                     timestamps.astype(jnp.int32))


# speedup vs baseline: 2.0691x; 1.1130x over previous
import functools

import jax
import jax.numpy as jnp
from jax import lax
from jax.experimental import pallas as pl
from jax.experimental.pallas import tpu as pltpu
from jax.experimental.pallas import tpu_sc as plsc

BATCH = 16384
ENT_D = 128
REL_D = 64
NUM_BUCKETS = 52
SECONDS_PER_WEEK = 7 * 86400

NUM_CORES = 2
NUM_SUBCORES = 16
NUM_WORKERS = NUM_CORES * NUM_SUBCORES
ROWS_PER_WORKER = BATCH // NUM_WORKERS
CHUNK = 64
NUM_CHUNKS = ROWS_PER_WORKER // CHUNK
NUM_PAIRS = NUM_CHUNKS // 2
LANES = 16
GROUPS = CHUNK // LANES
NVEC = ENT_D // LANES
INV_WEEK = 1.0 / SECONDS_PER_WEEK


def _rel_cos_sin(relation_table):
    def body(r_ref, o_ref):
        r = r_ref[...]
        o_ref[:, :REL_D] = jnp.cos(r)
        o_ref[:, REL_D:] = jnp.sin(r)

    return pl.pallas_call(
        body,
        out_shape=jax.ShapeDtypeStruct((relation_table.shape[0], ENT_D),
                                       jnp.float32),
    )(relation_table)


def _rsqrt(m):
    i = lax.bitcast_convert_type(m, jnp.int32)
    i = jnp.int32(0x5F3759DF) - lax.shift_right_arithmetic(i, 1)
    y = lax.bitcast_convert_type(i, jnp.float32)
    y = y * (jnp.float32(1.5) - (m * jnp.float32(0.5)) * y * y)
    return y


def _sc_score(entity_table, rel_cs_table, time_normals,
              head_idx, tail_idx, relation_idx, timestamps):
    mesh = plsc.VectorSubcoreMesh(core_axis_name="c", subcore_axis_name="s")

    buf = lambda: [
        pltpu.VMEM((CHUNK,), jnp.int32),
        pltpu.VMEM((CHUNK,), jnp.int32),
        pltpu.VMEM((CHUNK,), jnp.int32),
        pltpu.VMEM((CHUNK,), jnp.int32),
        pltpu.VMEM((CHUNK,), jnp.int32),
        pltpu.VMEM((CHUNK, ENT_D), jnp.float32),
        pltpu.VMEM((CHUNK, ENT_D), jnp.float32),
        pltpu.VMEM((CHUNK, ENT_D), jnp.float32),
        pltpu.VMEM((CHUNK, ENT_D), jnp.float32),
        pltpu.SemaphoreType.DMA,
        pltpu.SemaphoreType.DMA,
    ]

    @functools.partial(
        pl.kernel,
        mesh=mesh,
        out_type=jax.ShapeDtypeStruct((BATCH,), jnp.float32),
        compiler_params=pltpu.CompilerParams(needs_layout_passes=False),
        scratch_types=buf() + buf() + [
            pltpu.VMEM((ROWS_PER_WORKER,), jnp.float32),
        ],
    )
    def score_kernel(ent_hbm, rel_hbm, tn_hbm, hidx_hbm, tidx_hbm, ridx_hbm,
                     ts_hbm, out_hbm, *bufs):
        (ha, ta, ra, tsa, ba, h_a, t_a, r_a, n_a, sema, isema,
         hb, tb, rb, tsb, bb, h_b, t_b, r_b, n_b, semb, isemb,
         out_v) = bufs
        buf_a = (ha, ta, ra, tsa, ba, h_a, t_a, r_a, n_a, sema, isema)
        buf_b = (hb, tb, rb, tsb, bb, h_b, t_b, r_b, n_b, semb, isemb)
        wid = lax.axis_index("s") * NUM_CORES + lax.axis_index("c")
        lane_iota = lax.iota(jnp.int32, LANES)

        def fire(cidx, b):
            (hidx_v, tidx_v, ridx_v, ts_v, bidx_v, h_v, t_v, r_v, n_v,
             sem, isem) = b
            base = wid * ROWS_PER_WORKER + cidx * CHUNK
            pltpu.async_copy(hidx_hbm.at[pl.ds(base, CHUNK)], hidx_v, isem)
            pltpu.async_copy(tidx_hbm.at[pl.ds(base, CHUNK)], tidx_v, isem)
            pltpu.async_copy(ridx_hbm.at[pl.ds(base, CHUNK)], ridx_v, isem)
            pltpu.async_copy(ts_hbm.at[pl.ds(base, CHUNK)], ts_v, isem)
            pltpu.make_async_copy(hidx_hbm.at[pl.ds(base, CHUNK)],
                                  hidx_v, isem).wait()
            pltpu.make_async_copy(tidx_hbm.at[pl.ds(base, CHUNK)],
                                  tidx_v, isem).wait()
            pltpu.make_async_copy(ridx_hbm.at[pl.ds(base, CHUNK)],
                                  ridx_v, isem).wait()
            pltpu.make_async_copy(ts_hbm.at[pl.ds(base, CHUNK)],
                                  ts_v, isem).wait()
            for jj in range(GROUPS):
                ts16 = ts_v[pl.ds(jj * LANES, LANES)]
                q = (ts16.astype(jnp.float32)
                     * jnp.float32(INV_WEEK)).astype(jnp.int32)
                rem = ts16 - q * SECONDS_PER_WEEK
                q = q + jnp.where(rem < 0, -1, 0)
                q = q + jnp.where(rem >= SECONDS_PER_WEEK, 1, 0)
                bidx_v[pl.ds(jj * LANES, LANES)] = jnp.minimum(
                    q, NUM_BUCKETS - 1)
            pltpu.async_copy(ent_hbm.at[hidx_v], h_v, sem)
            pltpu.async_copy(ent_hbm.at[tidx_v], t_v, sem)
            pltpu.async_copy(rel_hbm.at[ridx_v], r_v, sem)
            pltpu.async_copy(tn_hbm.at[bidx_v], n_v, sem)

        def drain(b):
            (hidx_v, tidx_v, ridx_v, ts_v, bidx_v, h_v, t_v, r_v, n_v,
             sem, isem) = b
            pltpu.make_async_copy(ent_hbm.at[hidx_v], h_v, sem).wait()
            pltpu.make_async_copy(ent_hbm.at[tidx_v], t_v, sem).wait()
            pltpu.make_async_copy(rel_hbm.at[ridx_v], r_v, sem).wait()
            pltpu.make_async_copy(tn_hbm.at[bidx_v], n_v, sem).wait()

        def compute(cidx, b):
            _, _, _, _, _, h_v, t_v, r_v, n_v, _, _ = b

            def group_body(j, _):
                acc_out = jnp.zeros((LANES,), jnp.float32)
                for e in range(LANES):
                    i = j * LANES + e
                    dph_vec = jnp.zeros((LANES,), jnp.float32)
                    dpt_vec = jnp.zeros((LANES,), jnp.float32)
                    for k in range(NVEC):
                        sl = pl.ds(k * LANES, LANES)
                        n = n_v[i, sl]
                        dph_vec = dph_vec + h_v[i, sl] * n
                        dpt_vec = dpt_vec + t_v[i, sl] * n
                    dph = jnp.sum(dph_vec)
                    dpt = jnp.sum(dpt_vec)
                    acc = jnp.zeros((LANES,), jnp.float32)
                    for k in range(NVEC // 2):
                        slr = pl.ds(k * LANES, LANES)
                        sli = pl.ds(REL_D + k * LANES, LANES)
                        nre = n_v[i, slr]
                        nim = n_v[i, sli]
                        hpre = h_v[i, slr] - dph * nre
                        hpim = h_v[i, sli] - dph * nim
                        tpre = t_v[i, slr] - dpt * nre
                        tpim = t_v[i, sli] - dpt * nim
                        c = r_v[i, slr]
                        s = r_v[i, sli]
                        re_s = hpre * c - hpim * s - tpre
                        im_s = hpre * s + hpim * c - tpim
                        m = re_s * re_s + im_s * im_s + jnp.float32(1e-12)
                        acc = acc + m * _rsqrt(m)
                    total = jnp.sum(acc)
                    acc_out = jnp.where(lane_iota == e, -total, acc_out)
                out_v[pl.ds(cidx * CHUNK + j * LANES, LANES)] = acc_out
                return 0

            lax.fori_loop(0, GROUPS, group_body, 0, unroll=False)

        fire(0, buf_a)

        def pair_body(p, _):
            fire(2 * p + 1, buf_b)
            drain(buf_a)
            compute(2 * p, buf_a)

            @pl.when(p < NUM_PAIRS - 1)
            def _():
                fire(2 * p + 2, buf_a)

            drain(buf_b)
            compute(2 * p + 1, buf_b)
            return 0

        lax.fori_loop(0, NUM_PAIRS, pair_body, 0, unroll=False)
        pltpu.sync_copy(out_v, out_hbm.at[pl.ds(wid * ROWS_PER_WORKER,
                                                ROWS_PER_WORKER)])

    return score_kernel(entity_table, rel_cs_table, time_normals,
                        head_idx, tail_idx, relation_idx, timestamps)


def kernel(head_idx, relation_idx, tail_idx, timestamps,
           entity_table, relation_table, time_normals):
    rel_cs = _rel_cos_sin(relation_table)
    return _sc_score(entity_table, rel_cs, time_normals,
                     head_idx.astype(jnp.int32), tail_idx.astype(jnp.int32),
                     relation_idx.astype(jnp.int32),
                     timestamps.astype(jnp.int32))


# speedup vs baseline: 2.2474x; 1.0862x over previous
import functools

import jax
import jax.numpy as jnp
from jax import lax
from jax.experimental import pallas as pl
from jax.experimental.pallas import tpu as pltpu
from jax.experimental.pallas import tpu_sc as plsc

BATCH = 16384
ENT_D = 128
REL_D = 64
NUM_BUCKETS = 52
SECONDS_PER_WEEK = 7 * 86400

NUM_CORES = 2
NUM_SUBCORES = 16
NUM_WORKERS = NUM_CORES * NUM_SUBCORES
ROWS_PER_WORKER = BATCH // NUM_WORKERS
CHUNK = 64
NUM_CHUNKS = ROWS_PER_WORKER // CHUNK
LANES = 16
GROUPS = CHUNK // LANES
NVEC = ENT_D // LANES
INV_WEEK = 1.0 / SECONDS_PER_WEEK


def _rel_cos_sin(relation_table):
    def body(r_ref, o_ref):
        r = r_ref[...]
        o_ref[:, :REL_D] = jnp.cos(r)
        o_ref[:, REL_D:] = jnp.sin(r)

    return pl.pallas_call(
        body,
        out_shape=jax.ShapeDtypeStruct((relation_table.shape[0], ENT_D),
                                       jnp.float32),
    )(relation_table)


def _rsqrt(m):
    i = lax.bitcast_convert_type(m, jnp.int32)
    i = jnp.int32(0x5F3759DF) - lax.shift_right_arithmetic(i, 1)
    y = lax.bitcast_convert_type(i, jnp.float32)
    y = y * (jnp.float32(1.5) - (m * jnp.float32(0.5)) * y * y)
    return y


def _sc_score(entity_table, rel_cs_table, time_normals,
              head_idx, tail_idx, relation_idx, timestamps):
    mesh = plsc.VectorSubcoreMesh(core_axis_name="c", subcore_axis_name="s")

    @functools.partial(
        pl.kernel,
        mesh=mesh,
        out_type=jax.ShapeDtypeStruct((BATCH,), jnp.float32),
        compiler_params=pltpu.CompilerParams(needs_layout_passes=False),
        scratch_types=[
            pltpu.VMEM((2, CHUNK), jnp.int32),
            pltpu.VMEM((2, CHUNK), jnp.int32),
            pltpu.VMEM((2, CHUNK), jnp.int32),
            pltpu.VMEM((2, CHUNK), jnp.int32),
            pltpu.VMEM((2, CHUNK), jnp.int32),
            pltpu.VMEM((2, CHUNK, ENT_D), jnp.float32),
            pltpu.VMEM((2, CHUNK, ENT_D), jnp.float32),
            pltpu.VMEM((2, CHUNK, ENT_D), jnp.float32),
            pltpu.VMEM((2, CHUNK, ENT_D), jnp.float32),
            pltpu.SemaphoreType.DMA,
            pltpu.SemaphoreType.DMA,
            pltpu.SemaphoreType.DMA,
            pltpu.SemaphoreType.DMA,
            pltpu.VMEM((ROWS_PER_WORKER,), jnp.float32),
        ],
    )
    def score_kernel(ent_hbm, rel_hbm, tn_hbm, hidx_hbm, tidx_hbm, ridx_hbm,
                     ts_hbm, out_hbm,
                     hidx_v, tidx_v, ridx_v, ts_v, bidx_v, h_v, t_v, r_v,
                     n_v, sem0, sem1, isem0, isem1, out_v):
        wid = lax.axis_index("s") * NUM_CORES + lax.axis_index("c")
        lane_iota = lax.iota(jnp.int32, LANES)

        def slot(k):
            return (hidx_v.at[k], tidx_v.at[k], ridx_v.at[k], ts_v.at[k],
                    bidx_v.at[k], h_v.at[k], t_v.at[k], r_v.at[k], n_v.at[k],
                    (sem0, sem1)[k], (isem0, isem1)[k])

        def fire(cidx, b):
            (hidx, tidx, ridx, ts, bidx, h, t, r, n, sem, isem) = b
            base = wid * ROWS_PER_WORKER + cidx * CHUNK
            pltpu.async_copy(hidx_hbm.at[pl.ds(base, CHUNK)], hidx, isem)
            pltpu.async_copy(tidx_hbm.at[pl.ds(base, CHUNK)], tidx, isem)
            pltpu.async_copy(ridx_hbm.at[pl.ds(base, CHUNK)], ridx, isem)
            pltpu.async_copy(ts_hbm.at[pl.ds(base, CHUNK)], ts, isem)
            pltpu.make_async_copy(hidx_hbm.at[pl.ds(base, CHUNK)],
                                  hidx, isem).wait()
            pltpu.make_async_copy(tidx_hbm.at[pl.ds(base, CHUNK)],
                                  tidx, isem).wait()
            pltpu.make_async_copy(ridx_hbm.at[pl.ds(base, CHUNK)],
                                  ridx, isem).wait()
            pltpu.make_async_copy(ts_hbm.at[pl.ds(base, CHUNK)],
                                  ts, isem).wait()
            for jj in range(GROUPS):
                ts16 = ts[pl.ds(jj * LANES, LANES)]
                q = (ts16.astype(jnp.float32)
                     * jnp.float32(INV_WEEK)).astype(jnp.int32)
                rem = ts16 - q * SECONDS_PER_WEEK
                q = q + jnp.where(rem < 0, -1, 0)
                q = q + jnp.where(rem >= SECONDS_PER_WEEK, 1, 0)
                bidx[pl.ds(jj * LANES, LANES)] = jnp.minimum(
                    q, NUM_BUCKETS - 1)
            pltpu.async_copy(ent_hbm.at[hidx], h, sem)
            pltpu.async_copy(ent_hbm.at[tidx], t, sem)
            pltpu.async_copy(rel_hbm.at[ridx], r, sem)
            pltpu.async_copy(tn_hbm.at[bidx], n, sem)

        def drain(b):
            (hidx, tidx, ridx, ts, bidx, h, t, r, n, sem, isem) = b
            pltpu.make_async_copy(ent_hbm.at[hidx], h, sem).wait()
            pltpu.make_async_copy(ent_hbm.at[tidx], t, sem).wait()
            pltpu.make_async_copy(rel_hbm.at[ridx], r, sem).wait()
            pltpu.make_async_copy(tn_hbm.at[bidx], n, sem).wait()

        fire(0, slot(0))

        def chunk_body(cidx, _):
            parity = lax.bitwise_and(cidx, 1)
            more = cidx < NUM_CHUNKS - 1

            @pl.when(jnp.logical_and(more, parity == 0))
            def _():
                fire(cidx + 1, slot(1))

            @pl.when(jnp.logical_and(more, parity == 1))
            def _():
                fire(cidx + 1, slot(0))

            @pl.when(parity == 0)
            def _():
                drain(slot(0))

            @pl.when(parity == 1)
            def _():
                drain(slot(1))

            def group_body(j, _):
                acc_out = jnp.zeros((LANES,), jnp.float32)
                for e in range(LANES):
                    i = j * LANES + e
                    dph_vec = jnp.zeros((LANES,), jnp.float32)
                    dpt_vec = jnp.zeros((LANES,), jnp.float32)
                    for k in range(NVEC):
                        sl = pl.ds(k * LANES, LANES)
                        n = n_v[parity, i, sl]
                        dph_vec = dph_vec + h_v[parity, i, sl] * n
                        dpt_vec = dpt_vec + t_v[parity, i, sl] * n
                    dph = jnp.sum(dph_vec)
                    dpt = jnp.sum(dpt_vec)
                    acc = jnp.zeros((LANES,), jnp.float32)
                    for k in range(NVEC // 2):
                        slr = pl.ds(k * LANES, LANES)
                        sli = pl.ds(REL_D + k * LANES, LANES)
                        nre = n_v[parity, i, slr]
                        nim = n_v[parity, i, sli]
                        hpre = h_v[parity, i, slr] - dph * nre
                        hpim = h_v[parity, i, sli] - dph * nim
                        tpre = t_v[parity, i, slr] - dpt * nre
                        tpim = t_v[parity, i, sli] - dpt * nim
                        c = r_v[parity, i, slr]
                        s = r_v[parity, i, sli]
                        re_s = hpre * c - hpim * s - tpre
                        im_s = hpre * s + hpim * c - tpim
                        m = re_s * re_s + im_s * im_s + jnp.float32(1e-12)
                        acc = acc + m * _rsqrt(m)
                    total = jnp.sum(acc)
                    acc_out = jnp.where(lane_iota == e, -total, acc_out)
                out_v[pl.ds(cidx * CHUNK + j * LANES, LANES)] = acc_out
                return 0

            lax.fori_loop(0, GROUPS, group_body, 0, unroll=False)
            return 0

        lax.fori_loop(0, NUM_CHUNKS, chunk_body, 0, unroll=False)
        pltpu.sync_copy(out_v, out_hbm.at[pl.ds(wid * ROWS_PER_WORKER,
                                                ROWS_PER_WORKER)])

    return score_kernel(entity_table, rel_cs_table, time_normals,
                        head_idx, tail_idx, relation_idx, timestamps)


def kernel(head_idx, relation_idx, tail_idx, timestamps,
           entity_table, relation_table, time_normals):
    rel_cs = _rel_cos_sin(relation_table)
    return _sc_score(entity_table, rel_cs, time_normals,
                     head_idx.astype(jnp.int32), tail_idx.astype(jnp.int32),
                     relation_idx.astype(jnp.int32),
                     timestamps.astype(jnp.int32))


# speedup vs baseline: 2.3009x; 1.0238x over previous
import functools

import jax
import jax.numpy as jnp
from jax import lax
from jax.experimental import pallas as pl
from jax.experimental.pallas import tpu as pltpu
from jax.experimental.pallas import tpu_sc as plsc

BATCH = 16384
ENT_D = 128
REL_D = 64
NUM_BUCKETS = 52
SECONDS_PER_WEEK = 7 * 86400

NUM_CORES = 2
NUM_SUBCORES = 16
NUM_WORKERS = NUM_CORES * NUM_SUBCORES
ROWS_PER_WORKER = BATCH // NUM_WORKERS
CHUNK = 64
NUM_CHUNKS = ROWS_PER_WORKER // CHUNK
LANES = 16
GROUPS = CHUNK // LANES
NVEC = ENT_D // LANES
INV_WEEK = 1.0 / SECONDS_PER_WEEK


def _rel_cos_sin(relation_table):
    def body(r_ref, o_ref):
        r = r_ref[...]
        o_ref[:, :REL_D] = jnp.cos(r)
        o_ref[:, REL_D:] = jnp.sin(r)

    return pl.pallas_call(
        body,
        out_shape=jax.ShapeDtypeStruct((relation_table.shape[0], ENT_D),
                                       jnp.float32),
    )(relation_table)


def _rsqrt(m):
    i = lax.bitcast_convert_type(m, jnp.int32)
    i = jnp.int32(0x5F3759DF) - lax.shift_right_arithmetic(i, 1)
    y = lax.bitcast_convert_type(i, jnp.float32)
    y = y * (jnp.float32(1.5) - (m * jnp.float32(0.5)) * y * y)
    return y


def _sc_score(entity_table, rel_cs_table, time_normals,
              head_idx, tail_idx, relation_idx, timestamps):
    mesh = plsc.VectorSubcoreMesh(core_axis_name="c", subcore_axis_name="s")

    @functools.partial(
        pl.kernel,
        mesh=mesh,
        out_type=jax.ShapeDtypeStruct((BATCH,), jnp.float32),
        compiler_params=pltpu.CompilerParams(needs_layout_passes=False),
        scratch_types=[
            pltpu.VMEM((2, CHUNK), jnp.int32),
            pltpu.VMEM((2, CHUNK), jnp.int32),
            pltpu.VMEM((2, CHUNK), jnp.int32),
            pltpu.VMEM((2, CHUNK), jnp.int32),
            pltpu.VMEM((2, CHUNK), jnp.int32),
            pltpu.VMEM((2, CHUNK, ENT_D), jnp.float32),
            pltpu.VMEM((2, CHUNK, ENT_D), jnp.float32),
            pltpu.VMEM((2, CHUNK, ENT_D), jnp.float32),
            pltpu.VMEM((2, CHUNK, ENT_D), jnp.float32),
            pltpu.SemaphoreType.DMA,
            pltpu.SemaphoreType.DMA,
            pltpu.SemaphoreType.DMA,
            pltpu.SemaphoreType.DMA,
            pltpu.VMEM((ROWS_PER_WORKER,), jnp.float32),
        ],
    )
    def score_kernel(ent_hbm, rel_hbm, tn_hbm, hidx_hbm, tidx_hbm, ridx_hbm,
                     ts_hbm, out_hbm,
                     hidx_v, tidx_v, ridx_v, ts_v, bidx_v, h_v, t_v, r_v,
                     n_v, sem0, sem1, isem0, isem1, out_v):
        wid = lax.axis_index("s") * NUM_CORES + lax.axis_index("c")
        lane_iota = lax.iota(jnp.int32, LANES)

        def slot(k):
            return (hidx_v.at[k], tidx_v.at[k], ridx_v.at[k], ts_v.at[k],
                    bidx_v.at[k], h_v.at[k], t_v.at[k], r_v.at[k], n_v.at[k],
                    (sem0, sem1)[k], (isem0, isem1)[k])

        def fire_idx(cidx, b):
            (hidx, tidx, ridx, ts, bidx, h, t, r, n, sem, isem) = b
            base = wid * ROWS_PER_WORKER + cidx * CHUNK
            pltpu.async_copy(hidx_hbm.at[pl.ds(base, CHUNK)], hidx, isem)
            pltpu.async_copy(tidx_hbm.at[pl.ds(base, CHUNK)], tidx, isem)
            pltpu.async_copy(ridx_hbm.at[pl.ds(base, CHUNK)], ridx, isem)
            pltpu.async_copy(ts_hbm.at[pl.ds(base, CHUNK)], ts, isem)

        def fire_rows(cidx, b):
            (hidx, tidx, ridx, ts, bidx, h, t, r, n, sem, isem) = b
            base = wid * ROWS_PER_WORKER + cidx * CHUNK
            pltpu.make_async_copy(hidx_hbm.at[pl.ds(base, CHUNK)],
                                  hidx, isem).wait()
            pltpu.make_async_copy(tidx_hbm.at[pl.ds(base, CHUNK)],
                                  tidx, isem).wait()
            pltpu.make_async_copy(ridx_hbm.at[pl.ds(base, CHUNK)],
                                  ridx, isem).wait()
            pltpu.make_async_copy(ts_hbm.at[pl.ds(base, CHUNK)],
                                  ts, isem).wait()
            for jj in range(GROUPS):
                ts16 = ts[pl.ds(jj * LANES, LANES)]
                q = (ts16.astype(jnp.float32)
                     * jnp.float32(INV_WEEK)).astype(jnp.int32)
                rem = ts16 - q * SECONDS_PER_WEEK
                q = q + jnp.where(rem < 0, -1, 0)
                q = q + jnp.where(rem >= SECONDS_PER_WEEK, 1, 0)
                bidx[pl.ds(jj * LANES, LANES)] = jnp.minimum(
                    q, NUM_BUCKETS - 1)
            pltpu.async_copy(ent_hbm.at[hidx], h, sem)
            pltpu.async_copy(ent_hbm.at[tidx], t, sem)
            pltpu.async_copy(rel_hbm.at[ridx], r, sem)
            pltpu.async_copy(tn_hbm.at[bidx], n, sem)

        def drain(b):
            (hidx, tidx, ridx, ts, bidx, h, t, r, n, sem, isem) = b
            pltpu.make_async_copy(ent_hbm.at[hidx], h, sem).wait()
            pltpu.make_async_copy(ent_hbm.at[tidx], t, sem).wait()
            pltpu.make_async_copy(rel_hbm.at[ridx], r, sem).wait()
            pltpu.make_async_copy(tn_hbm.at[bidx], n, sem).wait()

        fire_idx(0, slot(0))
        fire_rows(0, slot(0))

        def chunk_body(cidx, _):
            parity = lax.bitwise_and(cidx, 1)
            more = cidx < NUM_CHUNKS - 1

            @pl.when(jnp.logical_and(more, parity == 0))
            def _():
                fire_idx(cidx + 1, slot(1))

            @pl.when(jnp.logical_and(more, parity == 1))
            def _():
                fire_idx(cidx + 1, slot(0))

            @pl.when(parity == 0)
            def _():
                drain(slot(0))

            @pl.when(parity == 1)
            def _():
                drain(slot(1))

            @pl.when(jnp.logical_and(more, parity == 0))
            def _():
                fire_rows(cidx + 1, slot(1))

            @pl.when(jnp.logical_and(more, parity == 1))
            def _():
                fire_rows(cidx + 1, slot(0))

            def group_body(j, _):
                acc_out = jnp.zeros((LANES,), jnp.float32)
                for e in range(LANES):
                    i = j * LANES + e
                    dph_vec = jnp.zeros((LANES,), jnp.float32)
                    dpt_vec = jnp.zeros((LANES,), jnp.float32)
                    for k in range(NVEC):
                        sl = pl.ds(k * LANES, LANES)
                        n = n_v[parity, i, sl]
                        dph_vec = dph_vec + h_v[parity, i, sl] * n
                        dpt_vec = dpt_vec + t_v[parity, i, sl] * n
                    dph = jnp.sum(dph_vec)
                    dpt = jnp.sum(dpt_vec)
                    acc = jnp.zeros((LANES,), jnp.float32)
                    for k in range(NVEC // 2):
                        slr = pl.ds(k * LANES, LANES)
                        sli = pl.ds(REL_D + k * LANES, LANES)
                        nre = n_v[parity, i, slr]
                        nim = n_v[parity, i, sli]
                        hpre = h_v[parity, i, slr] - dph * nre
                        hpim = h_v[parity, i, sli] - dph * nim
                        tpre = t_v[parity, i, slr] - dpt * nre
                        tpim = t_v[parity, i, sli] - dpt * nim
                        c = r_v[parity, i, slr]
                        s = r_v[parity, i, sli]
                        re_s = hpre * c - hpim * s - tpre
                        im_s = hpre * s + hpim * c - tpim
                        m = re_s * re_s + im_s * im_s + jnp.float32(1e-12)
                        acc = acc + m * _rsqrt(m)
                    total = jnp.sum(acc)
                    acc_out = jnp.where(lane_iota == e, -total, acc_out)
                out_v[pl.ds(cidx * CHUNK + j * LANES, LANES)] = acc_out
                return 0

            lax.fori_loop(0, GROUPS, group_body, 0, unroll=False)
            return 0

        lax.fori_loop(0, NUM_CHUNKS, chunk_body, 0, unroll=False)
        pltpu.sync_copy(out_v, out_hbm.at[pl.ds(wid * ROWS_PER_WORKER,
                                                ROWS_PER_WORKER)])

    return score_kernel(entity_table, rel_cs_table, time_normals,
                        head_idx, tail_idx, relation_idx, timestamps)


def kernel(head_idx, relation_idx, tail_idx, timestamps,
           entity_table, relation_table, time_normals):
    rel_cs = _rel_cos_sin(relation_table)
    return _sc_score(entity_table, rel_cs, time_normals,
                     head_idx.astype(jnp.int32), tail_idx.astype(jnp.int32),
                     relation_idx.astype(jnp.int32),
                     timestamps.astype(jnp.int32))
